# Initial kernel scaffold; baseline (speedup 1.0000x reference)
import jax
import jax.numpy as jnp
from jax.experimental import pallas as pl


def kernel(x, pos, edge_index, batch, params):
    raise NotImplementedError("write your pallas kernel here")


# speedup vs baseline: 4.0295x; 4.0295x over previous
import functools

import jax
import jax.numpy as jnp
from jax import lax
from jax.experimental import pallas as pl
from jax.experimental.pallas import tpu as pltpu
from jax.experimental.pallas import tpu_sc as plsc

_NC = 2
_NS = 16
_NW = _NC * _NS
_CH = 128
_BN = 1000
_BE = 8000
_G = 64


def _prep_kernel(x_ref, pos_ref, wn0, bn0, wn1, bn1, w0x, b0, a_ref, t_ref):
    xb = x_ref[...]
    h = jnp.dot(jax.nn.relu(jnp.dot(xb, wn0[...]) + bn0[...]), wn1[...]) + bn1[...]
    p4 = pos_ref[...]
    nrm = jnp.sqrt(jnp.sum(p4 * p4, axis=1, keepdims=True))
    n4 = p4 / (nrm + 1e-12)
    t_ref[:, 0:4] = p4
    t_ref[:, 4:8] = n4
    t_ref[:, 8:16] = jnp.zeros_like(t_ref[:, 8:16])
    a_ref[...] = jnp.dot(h, w0x[...]) + b0[...]


def _wprep_kernel(wl1_ref, wg0_ref, bl1_ref, m_ref, c_ref):
    wg0 = wg0_ref[0]
    m_ref[0] = jnp.dot(wl1_ref[0], wg0)
    c_ref[0] = jnp.dot(bl1_ref[0], wg0)


def _feat_kernel(ps_ref, pd_ref, f_ref):
    S = ps_ref[...].T
    D = pd_ref[...].T
    ps, ns = S[0:3, :], S[4:7, :]
    pd, nd = D[0:3, :], D[4:7, :]
    pseudo = ps - pd

    def roll1(v):
        return jnp.concatenate([v[1:3], v[0:1]], axis=0)

    def ang(v1, v2):
        a1, b1 = roll1(v1), roll1(v2)
        a2, b2 = roll1(a1), roll1(b1)
        c = a1 * b2 - a2 * b1
        cn = jnp.sqrt(jnp.sum(c * c, axis=0, keepdims=True))
        dt = jnp.sum(v1 * v2, axis=0, keepdims=True)
        return jnp.arctan2(cn, dt)

    f0 = jnp.sqrt(jnp.sum(pseudo * pseudo, axis=0, keepdims=True))
    pad = jnp.zeros((12, f0.shape[1]), jnp.float32)
    f_ref[...] = jnp.concatenate(
        [f0, ang(nd, pseudo), ang(ns, pseudo), ang(nd, ns), pad], axis=0).T


def _node_update(s, deg, m, cvec, bg0, wg1, bg1):
    u = jnp.dot(s, m) + deg * cvec + bg0
    return jax.nn.relu(jnp.dot(jax.nn.relu(u), wg1) + bg1)


def _layer0_kernel(sp_ref, m_ref, c_ref, bg0_ref, wg1_ref, bg1_ref,
                   w0x_ref, b0_ref, a_ref, deg8_ref):
    sw = sp_ref[0] + sp_ref[1]
    h = _node_update(sw[:, :128], sw[:, 128:129], m_ref[...], c_ref[...],
                     bg0_ref[...], wg1_ref[...], bg1_ref[...])
    a_ref[...] = jnp.dot(h, w0x_ref[...]) + b0_ref[...]
    deg8_ref[...] = sw[:, 128:136]


def _layer1_kernel(sp_ref, deg8_ref, m_ref, c_ref, bg0_ref, wg1_ref, bg1_ref,
                   w0x_ref, b0_ref, a_ref):
    s = sp_ref[0] + sp_ref[1]
    h = _node_update(s, deg8_ref[:, 0:1], m_ref[...], c_ref[...],
                     bg0_ref[...], wg1_ref[...], bg1_ref[...])
    a_ref[...] = jnp.dot(h, w0x_ref[...]) + b0_ref[...]


def _layer2_kernel(sp_ref, deg8_ref, m_ref, c_ref, bg0_ref, wg1_ref, bg1_ref,
                   w1r_ref, b1r_ref, w2r_ref, b2r_ref, batch_ref, out_ref):
    s = sp_ref[0] + sp_ref[1]
    h = _node_update(s, deg8_ref[:, 0:1], m_ref[...], c_ref[...],
                     bg0_ref[...], wg1_ref[...], bg1_ref[...])
    z = jax.nn.relu(jnp.dot(h, w1r_ref[...]) + b1r_ref[...])
    p = jnp.dot(z, w2r_ref[...]) + b2r_ref[...]
    oh = (batch_ref[...] == lax.broadcasted_iota(
        jnp.int32, (p.shape[0], _G), 1)).astype(jnp.float32)
    contrib = lax.dot_general(oh, p, (((0,), (0,)), ((), ())))

    @pl.when(pl.program_id(0) == 0)
    def _():
        out_ref[...] = contrib

    @pl.when(pl.program_id(0) != 0)
    def _():
        out_ref[...] += contrib


def _make_pair_gather(n, e):
    mesh = plsc.VectorSubcoreMesh(core_axis_name="c", subcore_axis_name="s")
    nchunk = e // _CH
    f32 = jnp.float32

    @functools.partial(
        pl.kernel,
        out_type=(jax.ShapeDtypeStruct((e, 16), f32),
                  jax.ShapeDtypeStruct((e, 16), f32)),
        mesh=mesh,
        compiler_params=pltpu.CompilerParams(use_tc_tiling_on_sc=False),
        scratch_types=[
            pltpu.VMEM((_CH,), jnp.int32),
            pltpu.VMEM((_CH,), jnp.int32),
            pltpu.VMEM((_CH, 16), f32),
            pltpu.VMEM((_CH, 16), f32),
            pltpu.SemaphoreType.DMA,
            pltpu.SemaphoreType.DMA,
        ],
    )
    def k(t_hbm, src_hbm, dst_hbm, ps_hbm, pd_hbm,
          sidx, didx, srows, drows, sem1, sem2):
        wid = lax.axis_index("s") * _NC + lax.axis_index("c")
        nch = (nchunk - wid + _NW - 1) // _NW

        def body(ci, carry):
            base = (wid + ci * _NW) * _CH
            pltpu.sync_copy(src_hbm.at[pl.ds(base, _CH)], sidx)
            pltpu.sync_copy(dst_hbm.at[pl.ds(base, _CH)], didx)
            cp1 = pltpu.async_copy(t_hbm.at[sidx], srows, sem1)
            cp2 = pltpu.async_copy(t_hbm.at[didx], drows, sem2)
            cp1.wait()
            cp2.wait()
            pltpu.sync_copy(srows, ps_hbm.at[pl.ds(base, _CH)])
            pltpu.sync_copy(drows, pd_hbm.at[pl.ds(base, _CH)])
            return carry

        lax.fori_loop(0, nch, body, 0)

    return k


def _make_edge(n, e, w):
    mesh = plsc.VectorSubcoreMesh(core_axis_name="c", subcore_axis_name="s")
    nchunk = e // _CH
    nzchunk = n // 16
    f32 = jnp.float32

    @functools.partial(
        pl.kernel,
        out_type=jax.ShapeDtypeStruct((_NC, n, w), f32),
        mesh=mesh,
        compiler_params=pltpu.CompilerParams(use_tc_tiling_on_sc=False),
        scratch_types=[
            pltpu.VMEM_SHARED((n, w), f32),
            pltpu.VMEM((_CH,), jnp.int32),
            pltpu.VMEM((_CH,), jnp.int32),
            pltpu.VMEM((_CH, 16), f32),
            pltpu.VMEM((_CH, 128), f32),
            pltpu.VMEM((_CH, w), f32),
            pltpu.VMEM((16, w), f32),
            pltpu.VMEM((4, 128), f32),
            pltpu.SemaphoreType.DMA,
        ],
    )
    def k(a_hbm, src_hbm, dst_hbm, feat_hbm, w0f_hbm, out_hbm,
          s_sh, srcv, dstv, featv, agath, msg, zbuf, w0fv, sem):
        cid = lax.axis_index("c")
        sid = lax.axis_index("s")
        wid = sid * _NC + cid
        zero16 = jnp.zeros((16,), f32)
        one16 = jnp.ones((16,), f32)

        def zrow(r, carry):
            for j in range(w // 16):
                zbuf[r, pl.ds(j * 16, 16)] = zero16
            return carry

        lax.fori_loop(0, 16, zrow, 0)

        if w > 128:
            def orow(r, carry):
                for j in range(8, w // 16):
                    msg[r, pl.ds(j * 16, 16)] = one16
                return carry

            lax.fori_loop(0, _CH, orow, 0)

        pltpu.sync_copy(w0f_hbm, w0fv)
        zcnt = (nzchunk - sid + _NS - 1) // _NS

        def zbody(ci, carry):
            off = pl.multiple_of((sid + ci * _NS) * 16, 16)
            pltpu.sync_copy(zbuf, s_sh.at[pl.ds(off, 16)])
            return carry

        lax.fori_loop(0, zcnt, zbody, 0)
        plsc.subcore_barrier()

        wv = [[w0fv[r, pl.ds(j * 16, 16)] for j in range(8)] for r in range(4)]
        nch = (nchunk - wid + _NW - 1) // _NW

        def chunk(ci, carry):
            base = (wid + ci * _NW) * _CH
            pltpu.sync_copy(src_hbm.at[pl.ds(base, _CH)], srcv)
            pltpu.sync_copy(dst_hbm.at[pl.ds(base, _CH)], dstv)
            pltpu.sync_copy(feat_hbm.at[pl.ds(base, _CH)], featv)
            pltpu.async_copy(a_hbm.at[srcv], agath, sem).wait()

            def ebody(i, ecarry):
                fv = featv[i, pl.ds(0, 16)]
                f0 = fv[0]
                f1 = fv[1]
                f2 = fv[2]
                f3 = fv[3]
                for j in range(8):
                    a = agath[i, pl.ds(j * 16, 16)]
                    cv = (wv[0][j] * f0 + wv[1][j] * f1
                          + wv[2][j] * f2 + wv[3][j] * f3)
                    msg[i, pl.ds(j * 16, 16)] = jnp.maximum(a + cv, 0.0)
                return ecarry

            lax.fori_loop(0, _CH, ebody, 0)
            pltpu.sync_copy(msg, s_sh.at[dstv], add=True)
            return carry

        lax.fori_loop(0, nch, chunk, 0)
        plsc.subcore_barrier()

        def obody(ci, carry):
            off = pl.multiple_of((sid + ci * _NS) * 16, 16)
            pltpu.sync_copy(s_sh.at[pl.ds(off, 16)],
                            out_hbm.at[cid, pl.ds(off, 16)])
            return carry

        lax.fori_loop(0, zcnt, obody, 0)

    return k


def _row(v):
    return v.reshape(1, -1)


def kernel(x, pos, edge_index, batch, params):
    n = x.shape[0]
    e = edge_index.shape[1]
    nb = n // _BN
    f32 = jnp.float32
    src = edge_index[0].astype(jnp.int32)
    dst = edge_index[1].astype(jnp.int32)
    pos4 = jnp.pad(pos.astype(f32), ((0, 0), (0, 1)))

    wn0, bn0, wn1, bn1 = params["node_lin"]
    loc = params["local"]
    glo = params["global"]
    w0x = [loc[i][0][:128] for i in range(3)]
    w0f = [loc[i][0][128:] for i in range(3)]
    b0l = [loc[i][1] for i in range(3)]
    w1r, b1r = params["lin1"]
    w2r, b2r = params["lin2"]

    full = lambda shp: pl.BlockSpec(shp, lambda i: tuple(0 for _ in shp))
    nblk = lambda shp: pl.BlockSpec(shp, lambda i: (i,) + tuple(0 for _ in shp[1:]))

    a0, tbl = pl.pallas_call(
        _prep_kernel,
        grid=(nb,),
        in_specs=[nblk((_BN, 128)), nblk((_BN, 4)), full((128, 128)),
                  full((1, 128)), full((128, 128)), full((1, 128)),
                  full((128, 128)), full((1, 128))],
        out_specs=[nblk((_BN, 128)), nblk((_BN, 16))],
        out_shape=[jax.ShapeDtypeStruct((n, 128), f32),
                   jax.ShapeDtypeStruct((n, 16), f32)],
    )(x, pos4, wn0, _row(bn0), wn1, _row(bn1), w0x[0], _row(b0l[0]))

    wl1s = jnp.stack([loc[i][2] for i in range(3)])
    wg0s = jnp.stack([glo[i][0] for i in range(3)])
    bl1s = jnp.stack([_row(loc[i][3]) for i in range(3)])
    ms, cs = pl.pallas_call(
        _wprep_kernel,
        grid=(3,),
        in_specs=[pl.BlockSpec((1, 128, 128), lambda i: (i, 0, 0)),
                  pl.BlockSpec((1, 128, 128), lambda i: (i, 0, 0)),
                  pl.BlockSpec((1, 1, 128), lambda i: (i, 0, 0))],
        out_specs=[pl.BlockSpec((1, 128, 128), lambda i: (i, 0, 0)),
                   pl.BlockSpec((1, 1, 128), lambda i: (i, 0, 0))],
        out_shape=[jax.ShapeDtypeStruct((3, 128, 128), f32),
                   jax.ShapeDtypeStruct((3, 1, 128), f32)],
    )(wl1s, wg0s, bl1s)

    prs, prd = _make_pair_gather(n, e)(tbl, src, dst)

    feat = pl.pallas_call(
        _feat_kernel,
        grid=(e // _BE,),
        in_specs=[nblk((_BE, 16)), nblk((_BE, 16))],
        out_specs=nblk((_BE, 16)),
        out_shape=jax.ShapeDtypeStruct((e, 16), f32),
    )(prs, prd)

    edge0 = _make_edge(n, e, 144)
    edge12 = _make_edge(n, e, 128)

    sp0 = edge0(a0, src, dst, feat, w0f[0])
    a1, deg8 = pl.pallas_call(
        _layer0_kernel,
        grid=(nb,),
        in_specs=[pl.BlockSpec((2, _BN, 144), lambda i: (0, i, 0)),
                  full((128, 128)), full((1, 128)), full((1, 128)),
                  full((128, 128)), full((1, 128)),
                  full((128, 128)), full((1, 128))],
        out_specs=[nblk((_BN, 128)), nblk((_BN, 8))],
        out_shape=[jax.ShapeDtypeStruct((n, 128), f32),
                   jax.ShapeDtypeStruct((n, 8), f32)],
    )(sp0, ms[0], cs[0], _row(glo[0][1]), glo[0][2], _row(glo[0][3]),
      w0x[1], _row(b0l[1]))

    sp1 = edge12(a1, src, dst, feat, w0f[1])
    a2 = pl.pallas_call(
        _layer1_kernel,
        grid=(nb,),
        in_specs=[pl.BlockSpec((2, _BN, 128), lambda i: (0, i, 0)),
                  nblk((_BN, 8)),
                  full((128, 128)), full((1, 128)), full((1, 128)),
                  full((128, 128)), full((1, 128)),
                  full((128, 128)), full((1, 128))],
        out_specs=nblk((_BN, 128)),
        out_shape=jax.ShapeDtypeStruct((n, 128), f32),
    )(sp1, deg8, ms[1], cs[1], _row(glo[1][1]), glo[1][2], _row(glo[1][3]),
      w0x[2], _row(b0l[2]))

    sp2 = edge12(a2, src, dst, feat, w0f[2])
    out = pl.pallas_call(
        _layer2_kernel,
        grid=(nb,),
        in_specs=[pl.BlockSpec((2, _BN, 128), lambda i: (0, i, 0)),
                  nblk((_BN, 8)),
                  full((128, 128)), full((1, 128)), full((1, 128)),
                  full((128, 128)), full((1, 128)),
                  full((128, 64)), full((1, 64)), full((64, 128)),
                  full((1, 128)), nblk((_BN, 1))],
        out_specs=pl.BlockSpec((_G, 128), lambda i: (0, 0)),
        out_shape=jax.ShapeDtypeStruct((_G, 128), f32),
    )(sp2, deg8, ms[2], cs[2], _row(glo[2][1]), glo[2][2], _row(glo[2][3]),
      w1r, _row(b1r), w2r, _row(b2r),
      batch.astype(jnp.int32).reshape(n, 1))
    return out
